# baseline (speedup 1.0000x reference)
import functools

import jax
import jax.numpy as jnp
from jax import lax
from jax.experimental import pallas as pl
from jax.experimental.pallas import tpu as pltpu
from jax.experimental.pallas import tpu_sc as plsc

B = 4
N_DET = 128
N_T = 2048
NY = 256
NX = 256
NPIX = NY * NX
N_TP = N_T + 1

NUM_CORES = 2
NUM_SUBCORES = 16
DETS_PER_W = N_DET // NUM_SUBCORES
NHALF = 4
PIX_H = NPIX // NHALF
PIX_W = PIX_H // NUM_CORES
PB = 512
PBB = 1024
PBM = 2048
PBF = 8192
_NBLK = PIX_W // PB
_BLK_PER_FLUSH = PBF // PB

_S_WORDS = DETS_PER_W * B * N_TP
_S_ALLOC = _S_WORDS + 16


def _norm_body(sino_ref, out_ref):
    x = sino_ref[0, 0]
    mean = jnp.mean(x)
    cent = x - mean
    var = jnp.mean(cent * cent)
    out_ref[0] = cent / jnp.sqrt(var + jnp.finfo(jnp.float32).eps)


def _lut_body(alpha_ref, valid_ref, k0_ref, lut_ref):
    a = alpha_ref[...]
    v = valid_ref[...]
    dets = lax.broadcasted_iota(jnp.int32, (1, N_DET), 1)
    dbase = (dets % DETS_PER_W) * (B * N_TP)
    k0e = jnp.where(v, k0_ref[...], N_T) + dbase
    aq = jnp.where(v, (a * 65536.0).astype(jnp.int32), 0)
    lut_ref[...] = (k0e | (aq << 16)).T


def _merge_body(p_ref, o_ref):
    o_ref[...] = jnp.sum(p_ref[0], axis=0)


def _das_sc_body(s_hbm, lut_hbm, apod_hbm, out_hbm, s_res, lutblk, accblk, apodbuf,
                 sem_l0, sem_l1):
    c = lax.axis_index("c")
    s = lax.axis_index("s")
    pixbase = c * PIX_W
    dbase = s * DETS_PER_W
    sems = (sem_l0, sem_l1)

    def lut_src(i):
        p0 = pixbase + i * PB
        return lut_hbm.at[pl.ds(dbase, DETS_PER_W), pl.ds(p0, PB)]

    def lut_start(i, slot):
        pltpu.async_copy(lut_src(i), lutblk.at[slot], sems[slot])

    def lut_wait(i, slot):
        pltpu.make_async_copy(lut_src(i), lutblk.at[slot], sems[slot]).wait()

    lut_start(0, 0)
    pltpu.sync_copy(apod_hbm, apodbuf.at[pl.ds(0, N_DET)])
    pltpu.sync_copy(s_hbm.at[pl.ds(s * _S_WORDS, _S_WORDS)],
                    s_res.at[pl.ds(0, _S_WORDS)])
    s_res[pl.ds(_S_WORDS, 16)] = jnp.zeros((16,), jnp.float32)

    asum = jnp.zeros((16,), jnp.float32)
    for i in range(N_DET // 16):
        asum = asum + apodbuf[pl.ds(i * 16, 16)]
    norm = jnp.maximum(jnp.sum(asum), jnp.finfo(jnp.float32).tiny)
    invv = jnp.ones((16,), jnp.float32) / jnp.full((16,), norm, jnp.float32)
    my_apod = apodbuf[pl.ds(dbase, 16)]
    av = []
    av16 = []
    for dl in range(DETS_PER_W):
        a_v = jnp.full((16,), my_apod[dl], jnp.float32) * invv
        av.append(a_v)
        av16.append(a_v * (1.0 / 65536.0))

    def compute_block(i, slot):
        off = (i % _BLK_PER_FLUSH) * PB

        def group_body(g, carry2):
            g16 = g * 16
            accs = [jnp.zeros((16,), jnp.float32) for _ in range(B)]
            for dl in range(DETS_PER_W):
                wv = lutblk[slot, dl, pl.ds(g16, 16)]
                k0v = jnp.bitwise_and(wv, jnp.int32(0xFFFF))
                aqf = lax.shift_right_logical(wv, 16).astype(jnp.float32)
                w1 = av16[dl] * aqf
                w0 = av[dl] - w1
                idx = k0v
                for b in range(B):
                    s0 = plsc.load_gather(s_res, [idx])
                    s1 = plsc.load_gather(s_res, [idx + 1])
                    accs[b] = accs[b] + w0 * s0
                    accs[b] = accs[b] + w1 * s1
                    if b < B - 1:
                        idx = idx + N_TP
            for b in range(B):
                accblk[b, pl.ds(off + g16, 16)] = accs[b]
            return carry2

        lax.fori_loop(0, PB // 16, group_body, 0)

    def pair_body(j, carry):
        b0 = 2 * j
        lut_start(b0 + 1, 1)
        lut_wait(b0, 0)
        compute_block(b0, 0)

        @pl.when(b0 + 2 < _NBLK)
        def _():
            lut_start(b0 + 2, 0)

        lut_wait(b0 + 1, 1)
        compute_block(b0 + 1, 1)

        @pl.when((b0 + 2) % _BLK_PER_FLUSH == 0)
        def _():
            q = (b0 + 2) // _BLK_PER_FLUSH - 1
            pltpu.sync_copy(accblk, out_hbm.at[c, s, :, pl.ds(q * PBF, PBF)])

        return carry

    lax.fori_loop(0, _NBLK // 2, pair_body, 0)


def kernel(sino, alpha, apod, k0, valid):
    s_n = pl.pallas_call(
        _norm_body,
        grid=(B,),
        in_specs=[pl.BlockSpec((1, 1, N_DET, N_T), lambda b: (b, 0, 0, 0))],
        out_specs=pl.BlockSpec((1, N_DET, N_T), lambda b: (b, 0, 0)),
        out_shape=jax.ShapeDtypeStruct((B, N_DET, N_T), jnp.float32),
    )(sino)
    s_flat = jnp.pad(jnp.transpose(s_n, (1, 0, 2)),
                     ((0, 0), (0, 0), (0, 1))).reshape(-1)

    a2 = alpha.reshape(NPIX, N_DET)
    v2 = valid.reshape(NPIX, N_DET)
    k2 = k0.reshape(NPIX, N_DET)
    lut_call = pl.pallas_call(
        _lut_body,
        grid=(PIX_H // PBB,),
        in_specs=[
            pl.BlockSpec((PBB, N_DET), lambda i: (i, 0)),
            pl.BlockSpec((PBB, N_DET), lambda i: (i, 0)),
            pl.BlockSpec((PBB, N_DET), lambda i: (i, 0)),
        ],
        out_specs=pl.BlockSpec((N_DET, PBB), lambda i: (0, i)),
        out_shape=jax.ShapeDtypeStruct((N_DET, PIX_H), jnp.int32),
    )

    mesh = plsc.VectorSubcoreMesh(core_axis_name="c", subcore_axis_name="s")
    das = functools.partial(
        pl.kernel,
        mesh=mesh,
        compiler_params=pltpu.CompilerParams(needs_layout_passes=False),
        out_type=jax.ShapeDtypeStruct((NUM_CORES, NUM_SUBCORES, B, PIX_W), jnp.float32),
        scratch_types=[
            pltpu.VMEM((_S_ALLOC,), jnp.float32),
            pltpu.VMEM((2, DETS_PER_W, PB), jnp.int32),
            pltpu.VMEM((B, PBF), jnp.float32),
            pltpu.VMEM((N_DET + 16,), jnp.float32),
            pltpu.SemaphoreType.DMA,
            pltpu.SemaphoreType.DMA,
        ],
    )(_das_sc_body)

    merge_call = pl.pallas_call(
        _merge_body,
        grid=(NUM_CORES, PIX_W // PBM),
        in_specs=[pl.BlockSpec((1, NUM_SUBCORES, B, PBM), lambda c, k: (c, 0, 0, k))],
        out_specs=pl.BlockSpec((B, PBM), lambda c, k: (0, c * (PIX_W // PBM) + k)),
        out_shape=jax.ShapeDtypeStruct((B, PIX_H), jnp.float32),
    )

    halves = []
    for h in range(NHALF):
        rows = slice(h * PIX_H, (h + 1) * PIX_H)
        lut_h = lut_call(a2[rows], v2[rows], k2[rows])
        partial_h = das(s_flat, lut_h, apod)
        halves.append(merge_call(partial_h))
    out = jnp.concatenate(halves, axis=1)
    return out.reshape(B, 1, NY, NX)


# speedup vs baseline: 1.0275x; 1.0275x over previous
import functools

import jax
import jax.numpy as jnp
from jax import lax
from jax.experimental import pallas as pl
from jax.experimental.pallas import tpu as pltpu
from jax.experimental.pallas import tpu_sc as plsc

B = 4
N_DET = 128
N_T = 2048
NY = 256
NX = 256
NPIX = NY * NX
N_TP = N_T + 1

NUM_CORES = 2
NUM_SUBCORES = 16
DETS_PER_W = N_DET // NUM_SUBCORES
NHALF = 2
PIX_H = NPIX // NHALF
PIX_W = PIX_H // NUM_CORES
PB = 1024
PBB = 1024
PBM = 2048
PBF = 4096
_NBLK = PIX_W // PB
_BLK_PER_FLUSH = PBF // PB
_NFLUSH = PIX_W // PBF

_S_WORDS = DETS_PER_W * B * N_TP
_S_ALLOC = _S_WORDS + 16


def _norm_body(sino_ref, out_ref):
    x = sino_ref[0, 0]
    mean = jnp.mean(x)
    cent = x - mean
    var = jnp.mean(cent * cent)
    out_ref[0] = cent / jnp.sqrt(var + jnp.finfo(jnp.float32).eps)


def _lut_body(alpha_ref, valid_ref, k0_ref, lut_ref):
    a = alpha_ref[...]
    v = valid_ref[...]
    dets = lax.broadcasted_iota(jnp.int32, (1, N_DET), 1)
    dbase = (dets % DETS_PER_W) * (B * N_TP)
    k0e = jnp.where(v, k0_ref[...], N_T) + dbase
    aq = jnp.where(v, (a * 65536.0).astype(jnp.int32), 0)
    lut_ref[...] = (k0e | (aq << 16)).T


def _merge_body(p_ref, o_ref):
    o_ref[...] = jnp.sum(p_ref[0], axis=0)


def _das_sc_body(s_hbm, lut_hbm, apod_hbm, out_hbm, s_res, lutblk, accblk, apodbuf,
                 sem_l0, sem_l1, sem_f0, sem_f1):
    c = lax.axis_index("c")
    s = lax.axis_index("s")
    pixbase = c * PIX_W
    dbase = s * DETS_PER_W
    sems = (sem_l0, sem_l1)
    fsems = (sem_f0, sem_f1)

    def lut_src(i):
        p0 = pixbase + i * PB
        return lut_hbm.at[pl.ds(dbase, DETS_PER_W), pl.ds(p0, PB)]

    def lut_start(i, slot):
        pltpu.async_copy(lut_src(i), lutblk.at[slot], sems[slot])

    def lut_wait(i, slot):
        pltpu.make_async_copy(lut_src(i), lutblk.at[slot], sems[slot]).wait()

    def flush_refs(q, par):
        return accblk.at[par], out_hbm.at[c, s, :, pl.ds(q * PBF, PBF)]

    def flush_start(q, par):
        src, dst = flush_refs(q, par)
        pltpu.async_copy(src, dst, fsems[par])

    def flush_wait(q, par):
        src, dst = flush_refs(q, par)
        pltpu.make_async_copy(src, dst, fsems[par]).wait()

    lut_start(0, 0)
    lut_start(1, 1)
    pltpu.sync_copy(apod_hbm, apodbuf.at[pl.ds(0, N_DET)])
    pltpu.sync_copy(s_hbm.at[pl.ds(s * _S_WORDS, _S_WORDS)],
                    s_res.at[pl.ds(0, _S_WORDS)])
    s_res[pl.ds(_S_WORDS, 16)] = jnp.zeros((16,), jnp.float32)

    asum = jnp.zeros((16,), jnp.float32)
    for i in range(N_DET // 16):
        asum = asum + apodbuf[pl.ds(i * 16, 16)]
    norm = jnp.maximum(jnp.sum(asum), jnp.finfo(jnp.float32).tiny)
    invv = jnp.ones((16,), jnp.float32) / jnp.full((16,), norm, jnp.float32)
    my_apod = apodbuf[pl.ds(dbase, 16)]
    av = []
    av16 = []
    for dl in range(DETS_PER_W):
        a_v = jnp.full((16,), my_apod[dl], jnp.float32) * invv
        av.append(a_v)
        av16.append(a_v * (1.0 / 65536.0))

    def compute_block(i, slot, par, off):
        def group_body(g, carry2):
            g16 = g * 16
            accs = [jnp.zeros((16,), jnp.float32) for _ in range(B)]
            for dl in range(DETS_PER_W):
                wv = lutblk[slot, dl, pl.ds(g16, 16)]
                k0v = jnp.bitwise_and(wv, jnp.int32(0xFFFF))
                aqf = lax.shift_right_logical(wv, 16).astype(jnp.float32)
                w1 = av16[dl] * aqf
                w0 = av[dl] - w1
                idx = k0v
                for b in range(B):
                    s0 = plsc.load_gather(s_res, [idx])
                    s1 = plsc.load_gather(s_res, [idx + 1])
                    accs[b] = accs[b] + w0 * s0
                    accs[b] = accs[b] + w1 * s1
                    if b < B - 1:
                        idx = idx + N_TP
            for b in range(B):
                accblk[par, b, pl.ds(off + g16, 16)] = accs[b]
            return carry2

        lax.fori_loop(0, PB // 16, group_body, 0)

    def superblock(ff, carry):
        for par in range(2):
            q = 2 * ff + par
            ibase = q * _BLK_PER_FLUSH

            @pl.when(q >= 2)
            def _(q=q, par=par):
                flush_wait(q - 2, par)

            for u in range(_BLK_PER_FLUSH):
                i = ibase + u
                sl = u % 2
                lut_wait(i, sl)
                compute_block(i, sl, par, u * PB)

                @pl.when(i + 2 < _NBLK)
                def _(i=i, sl=sl):
                    lut_start(i + 2, sl)

            flush_start(q, par)
        return carry

    lax.fori_loop(0, _NFLUSH // 2, superblock, 0)
    flush_wait(_NFLUSH - 2, 0)
    flush_wait(_NFLUSH - 1, 1)


def kernel(sino, alpha, apod, k0, valid):
    s_n = pl.pallas_call(
        _norm_body,
        grid=(B,),
        in_specs=[pl.BlockSpec((1, 1, N_DET, N_T), lambda b: (b, 0, 0, 0))],
        out_specs=pl.BlockSpec((1, N_DET, N_T), lambda b: (b, 0, 0)),
        out_shape=jax.ShapeDtypeStruct((B, N_DET, N_T), jnp.float32),
    )(sino)
    s_flat = jnp.pad(jnp.transpose(s_n, (1, 0, 2)),
                     ((0, 0), (0, 0), (0, 1))).reshape(-1)

    a2 = alpha.reshape(NPIX, N_DET)
    v2 = valid.reshape(NPIX, N_DET)
    k2 = k0.reshape(NPIX, N_DET)
    lut_call = pl.pallas_call(
        _lut_body,
        grid=(PIX_H // PBB,),
        in_specs=[
            pl.BlockSpec((PBB, N_DET), lambda i: (i, 0)),
            pl.BlockSpec((PBB, N_DET), lambda i: (i, 0)),
            pl.BlockSpec((PBB, N_DET), lambda i: (i, 0)),
        ],
        out_specs=pl.BlockSpec((N_DET, PBB), lambda i: (0, i)),
        out_shape=jax.ShapeDtypeStruct((N_DET, PIX_H), jnp.int32),
    )

    mesh = plsc.VectorSubcoreMesh(core_axis_name="c", subcore_axis_name="s")
    das = functools.partial(
        pl.kernel,
        mesh=mesh,
        compiler_params=pltpu.CompilerParams(needs_layout_passes=False),
        out_type=jax.ShapeDtypeStruct((NUM_CORES, NUM_SUBCORES, B, PIX_W), jnp.float32),
        scratch_types=[
            pltpu.VMEM((_S_ALLOC,), jnp.float32),
            pltpu.VMEM((2, DETS_PER_W, PB), jnp.int32),
            pltpu.VMEM((2, B, PBF), jnp.float32),
            pltpu.VMEM((N_DET + 16,), jnp.float32),
            pltpu.SemaphoreType.DMA,
            pltpu.SemaphoreType.DMA,
            pltpu.SemaphoreType.DMA,
            pltpu.SemaphoreType.DMA,
        ],
    )(_das_sc_body)

    merge_call = pl.pallas_call(
        _merge_body,
        grid=(NUM_CORES, PIX_W // PBM),
        in_specs=[pl.BlockSpec((1, NUM_SUBCORES, B, PBM), lambda c, k: (c, 0, 0, k))],
        out_specs=pl.BlockSpec((B, PBM), lambda c, k: (0, c * (PIX_W // PBM) + k)),
        out_shape=jax.ShapeDtypeStruct((B, PIX_H), jnp.float32),
    )

    halves = []
    for h in range(NHALF):
        rows = slice(h * PIX_H, (h + 1) * PIX_H)
        lut_h = lut_call(a2[rows], v2[rows], k2[rows])
        partial_h = das(s_flat, lut_h, apod)
        halves.append(merge_call(partial_h))
    out = jnp.concatenate(halves, axis=1)
    return out.reshape(B, 1, NY, NX)


# speedup vs baseline: 1.0414x; 1.0135x over previous
import functools

import jax
import jax.numpy as jnp
from jax import lax
from jax.experimental import pallas as pl
from jax.experimental.pallas import tpu as pltpu
from jax.experimental.pallas import tpu_sc as plsc

B = 4
N_DET = 128
N_T = 2048
NY = 256
NX = 256
NPIX = NY * NX
N_TP = N_T + 1

NUM_CORES = 2
NUM_SUBCORES = 16
DETS_PER_W = N_DET // NUM_SUBCORES
NHALF = 2
PIX_H = NPIX // NHALF
PIX_W = PIX_H // NUM_CORES
PB = 1024
PBB = 1024
PBM = 2048
PBF = 4096
_NBLK = PIX_W // PB
_BLK_PER_FLUSH = PBF // PB
_NFLUSH = PIX_W // PBF

_S_WORDS = DETS_PER_W * B * N_TP
_S_ALLOC = _S_WORDS + 16


def _norm_body(sino_ref, out_ref):
    x = sino_ref[:, 0]
    mean = jnp.mean(x, axis=(1, 2), keepdims=True)
    cent = x - mean
    var = jnp.mean(cent * cent, axis=(1, 2), keepdims=True)
    sn = cent / jnp.sqrt(var + jnp.finfo(jnp.float32).eps)
    out_ref[:, :, pl.ds(0, N_T)] = jnp.transpose(sn, (1, 0, 2))
    out_ref[:, :, pl.ds(N_T, 1)] = jnp.zeros((N_DET, B, 1), jnp.float32)


def _lut_body(alpha_ref, valid_ref, k0_ref, lut_ref):
    a = alpha_ref[...]
    v = valid_ref[...]
    dets = lax.broadcasted_iota(jnp.int32, (1, N_DET), 1)
    dbase = (dets % DETS_PER_W) * (B * N_TP)
    k0e = jnp.where(v, k0_ref[...], N_T) + dbase
    aq = jnp.where(v, (a * 65536.0).astype(jnp.int32), 0)
    lut_ref[...] = (k0e | (aq << 16)).T


def _merge_body(p_ref, o_ref):
    o_ref[...] = jnp.sum(p_ref[0], axis=0)


def _das_sc_body(s_hbm, lut_hbm, apod_hbm, out_hbm, s_res, lutblk, accblk, apodbuf,
                 sem_l0, sem_l1, sem_f0, sem_f1):
    c = lax.axis_index("c")
    s = lax.axis_index("s")
    pixbase = c * PIX_W
    dbase = s * DETS_PER_W
    sems = (sem_l0, sem_l1)
    fsems = (sem_f0, sem_f1)

    def lut_src(i):
        p0 = pixbase + i * PB
        return lut_hbm.at[pl.ds(dbase, DETS_PER_W), pl.ds(p0, PB)]

    def lut_start(i, slot):
        pltpu.async_copy(lut_src(i), lutblk.at[slot], sems[slot])

    def lut_wait(i, slot):
        pltpu.make_async_copy(lut_src(i), lutblk.at[slot], sems[slot]).wait()

    def flush_refs(q, par):
        return accblk.at[par], out_hbm.at[c, s, :, pl.ds(q * PBF, PBF)]

    def flush_start(q, par):
        src, dst = flush_refs(q, par)
        pltpu.async_copy(src, dst, fsems[par])

    def flush_wait(q, par):
        src, dst = flush_refs(q, par)
        pltpu.make_async_copy(src, dst, fsems[par]).wait()

    lut_start(0, 0)
    lut_start(1, 1)
    pltpu.sync_copy(apod_hbm, apodbuf.at[pl.ds(0, N_DET)])
    pltpu.sync_copy(s_hbm.at[pl.ds(s * _S_WORDS, _S_WORDS)],
                    s_res.at[pl.ds(0, _S_WORDS)])
    s_res[pl.ds(_S_WORDS, 16)] = jnp.zeros((16,), jnp.float32)

    asum = jnp.zeros((16,), jnp.float32)
    for i in range(N_DET // 16):
        asum = asum + apodbuf[pl.ds(i * 16, 16)]
    norm = jnp.maximum(jnp.sum(asum), jnp.finfo(jnp.float32).tiny)
    invv = jnp.ones((16,), jnp.float32) / jnp.full((16,), norm, jnp.float32)
    my_apod = apodbuf[pl.ds(dbase, 16)]
    av = []
    av16 = []
    for dl in range(DETS_PER_W):
        a_v = jnp.full((16,), my_apod[dl], jnp.float32) * invv
        av.append(a_v)
        av16.append(a_v * (1.0 / 65536.0))

    def compute_block(i, slot, par, off):
        def group_body(g, carry2):
            g16 = g * 16
            accs = [jnp.zeros((16,), jnp.float32) for _ in range(B)]
            for dl in range(DETS_PER_W):
                wv = lutblk[slot, dl, pl.ds(g16, 16)]
                k0v = jnp.bitwise_and(wv, jnp.int32(0xFFFF))
                aqf = lax.shift_right_logical(wv, 16).astype(jnp.float32)
                w1 = av16[dl] * aqf
                w0 = av[dl] - w1
                idx = k0v
                for b in range(B):
                    s0 = plsc.load_gather(s_res, [idx])
                    s1 = plsc.load_gather(s_res, [idx + 1])
                    accs[b] = accs[b] + w0 * s0
                    accs[b] = accs[b] + w1 * s1
                    if b < B - 1:
                        idx = idx + N_TP
            for b in range(B):
                accblk[par, b, pl.ds(off + g16, 16)] = accs[b]
            return carry2

        lax.fori_loop(0, PB // 16, group_body, 0)

    def superblock(ff, carry):
        for par in range(2):
            q = 2 * ff + par
            ibase = q * _BLK_PER_FLUSH

            @pl.when(q >= 2)
            def _(q=q, par=par):
                flush_wait(q - 2, par)

            for u in range(_BLK_PER_FLUSH):
                i = ibase + u
                sl = u % 2
                lut_wait(i, sl)
                compute_block(i, sl, par, u * PB)

                @pl.when(i + 2 < _NBLK)
                def _(i=i, sl=sl):
                    lut_start(i + 2, sl)

            flush_start(q, par)
        return carry

    lax.fori_loop(0, _NFLUSH // 2, superblock, 0)
    flush_wait(_NFLUSH - 2, 0)
    flush_wait(_NFLUSH - 1, 1)


def kernel(sino, alpha, apod, k0, valid):
    s_t = pl.pallas_call(
        _norm_body,
        out_shape=jax.ShapeDtypeStruct((N_DET, B, N_TP), jnp.float32),
    )(sino)
    s_flat = s_t.reshape(-1)

    a2 = alpha.reshape(NPIX, N_DET)
    v2 = valid.reshape(NPIX, N_DET)
    k2 = k0.reshape(NPIX, N_DET)
    lut_call = pl.pallas_call(
        _lut_body,
        grid=(PIX_H // PBB,),
        in_specs=[
            pl.BlockSpec((PBB, N_DET), lambda i: (i, 0)),
            pl.BlockSpec((PBB, N_DET), lambda i: (i, 0)),
            pl.BlockSpec((PBB, N_DET), lambda i: (i, 0)),
        ],
        out_specs=pl.BlockSpec((N_DET, PBB), lambda i: (0, i)),
        out_shape=jax.ShapeDtypeStruct((N_DET, PIX_H), jnp.int32),
    )

    mesh = plsc.VectorSubcoreMesh(core_axis_name="c", subcore_axis_name="s")
    das = functools.partial(
        pl.kernel,
        mesh=mesh,
        compiler_params=pltpu.CompilerParams(needs_layout_passes=False),
        out_type=jax.ShapeDtypeStruct((NUM_CORES, NUM_SUBCORES, B, PIX_W), jnp.float32),
        scratch_types=[
            pltpu.VMEM((_S_ALLOC,), jnp.float32),
            pltpu.VMEM((2, DETS_PER_W, PB), jnp.int32),
            pltpu.VMEM((2, B, PBF), jnp.float32),
            pltpu.VMEM((N_DET + 16,), jnp.float32),
            pltpu.SemaphoreType.DMA,
            pltpu.SemaphoreType.DMA,
            pltpu.SemaphoreType.DMA,
            pltpu.SemaphoreType.DMA,
        ],
    )(_das_sc_body)

    merge_call = pl.pallas_call(
        _merge_body,
        grid=(NUM_CORES, PIX_W // PBM),
        in_specs=[pl.BlockSpec((1, NUM_SUBCORES, B, PBM), lambda c, k: (c, 0, 0, k))],
        out_specs=pl.BlockSpec((B, PBM), lambda c, k: (0, c * (PIX_W // PBM) + k)),
        out_shape=jax.ShapeDtypeStruct((B, PIX_H), jnp.float32),
    )

    halves = []
    for h in range(NHALF):
        rows = slice(h * PIX_H, (h + 1) * PIX_H)
        lut_h = lut_call(a2[rows], v2[rows], k2[rows])
        partial_h = das(s_flat, lut_h, apod)
        halves.append(merge_call(partial_h))
    out = jnp.concatenate(halves, axis=1)
    return out.reshape(B, 1, NY, NX)
